# Initial kernel scaffold; baseline (speedup 1.0000x reference)
import jax
import jax.numpy as jnp
from jax.experimental import pallas as pl


def kernel(x, r_ij, neighbors, neighbor_mask, f_ij, W_in2f, b_in2f, W_G, W_res1, b_res1, W_res2, b_res2, W_res3, b_res3, W_dense, b_dense, mask):
    raise NotImplementedError("write your pallas kernel here")


# speedup vs baseline: 890.6571x; 890.6571x over previous
import functools

import jax
import jax.numpy as jnp
from jax import lax
from jax.experimental import pallas as pl
from jax.experimental.pallas import tpu as pltpu
from jax.experimental.pallas import tpu_sc as plsc

_LOG2 = 0.6931471805599453


def _ssp(v):
    return jnp.maximum(v, 0.0) + jnp.log1p(jnp.exp(-jnp.abs(v))) - _LOG2


def _tc_pre_body(x_ref, w_ref, b_ref, y_ref):
    v = _ssp(x_ref[...])
    v = jnp.dot(v, w_ref[...], preferred_element_type=jnp.float32) + b_ref[...]
    y_ref[...] = _ssp(v)


def _tc_pre(x2, W_in2f, b_in2f, blk):
    n, d = x2.shape
    grid = (n // blk,)
    return pl.pallas_call(
        _tc_pre_body,
        grid=grid,
        in_specs=[
            pl.BlockSpec((blk, d), lambda i: (i, 0)),
            pl.BlockSpec((d, d), lambda i: (0, 0)),
            pl.BlockSpec((1, d), lambda i: (0, 0)),
        ],
        out_specs=pl.BlockSpec((blk, d), lambda i: (i, 0)),
        out_shape=jax.ShapeDtypeStruct((n, d), jnp.float32),
    )(x2, W_in2f, b_in2f.reshape(1, d))


_NC, _NS = 2, 16
_NW = _NC * _NS
_CHUNK = 80


def _sc_gather(nbr3, y, n_edges, d):
    n_per_w = n_edges // _NW
    n_chunks = n_per_w // _CHUNK
    mesh = plsc.VectorSubcoreMesh(core_axis_name="c", subcore_axis_name="s")

    @functools.partial(
        pl.kernel,
        mesh=mesh,
        out_type=jax.ShapeDtypeStruct((n_edges, d), jnp.float32),
        scratch_types=[
            pltpu.VMEM((n_chunks, _CHUNK), jnp.int32),
            pltpu.VMEM((_CHUNK, d), jnp.float32),
            pltpu.SemaphoreType.DMA,
        ],
    )
    def gather_k(nbr_hbm, y_hbm, out_hbm, idx_v, buf_v, sem):
        wid = lax.axis_index("s") * _NC + lax.axis_index("c")
        base = wid * n_per_w
        pltpu.sync_copy(nbr_hbm.at[wid], idx_v)

        def body(i, carry):
            pltpu.async_copy(y_hbm.at[idx_v.at[i]], buf_v, sem).wait()
            pltpu.sync_copy(buf_v, out_hbm.at[pl.ds(base + i * _CHUNK, _CHUNK)])
            return carry

        lax.fori_loop(0, n_chunks, body, 0)

    return gather_k(nbr3, y)


def _tc_main_body(f_ref, g_ref, nm_ref, y_ref, x_ref,
                  wg_ref, w1_ref, b1_ref, w2_ref, b2_ref, w3_ref, b3_ref,
                  wd_ref, bd_ref, mask_ref, o_ref, *, blk, nbh):
    d = y_ref.shape[-1]
    sb = wg_ref.shape[0]
    f2 = f_ref[...].reshape(blk * nbh, sb)
    wf = jnp.dot(f2, wg_ref[...], preferred_element_type=jnp.float32)
    prod = g_ref[...] * wf * nm_ref[...]
    p3 = prod.reshape(blk, nbh, d)
    y2 = jnp.sum(p3, axis=1)
    y = y_ref[...] + y2
    h = y
    for w_r, b_r in ((w1_ref, b1_ref), (w2_ref, b2_ref), (w3_ref, b3_ref)):
        h = _ssp(h)
        h = jnp.dot(h, w_r[...], preferred_element_type=jnp.float32) + b_r[...]
    y = y + h
    y = _ssp(y)
    y = jnp.dot(y, wd_ref[...], preferred_element_type=jnp.float32) + bd_ref[...]
    o_ref[...] = y + mask_ref[...] * x_ref[...]


def _tc_main(f3, G, nm2, y, x2, W_G,
             W_res1, b_res1, W_res2, b_res2, W_res3, b_res3,
             W_dense, b_dense, mask, blk):
    n, d = x2.shape
    nbh = f3.shape[1]
    sb = f3.shape[2]
    grid = (n // blk,)
    w_spec = pl.BlockSpec((d, d), lambda i: (0, 0))
    b_spec = pl.BlockSpec((1, d), lambda i: (0, 0))
    return pl.pallas_call(
        functools.partial(_tc_main_body, blk=blk, nbh=nbh),
        grid=grid,
        in_specs=[
            pl.BlockSpec((blk, nbh, sb), lambda i: (i, 0, 0)),
            pl.BlockSpec((blk * nbh, d), lambda i: (i, 0)),
            pl.BlockSpec((blk * nbh, 1), lambda i: (i, 0)),
            pl.BlockSpec((blk, d), lambda i: (i, 0)),
            pl.BlockSpec((blk, d), lambda i: (i, 0)),
            pl.BlockSpec((sb, d), lambda i: (0, 0)),
            w_spec, b_spec, w_spec, b_spec, w_spec, b_spec,
            w_spec, b_spec, b_spec,
        ],
        out_specs=pl.BlockSpec((blk, d), lambda i: (i, 0)),
        out_shape=jax.ShapeDtypeStruct((n, d), jnp.float32),
    )(f3, G, nm2, y, x2, W_G,
      W_res1, b_res1.reshape(1, d), W_res2, b_res2.reshape(1, d),
      W_res3, b_res3.reshape(1, d), W_dense, b_dense.reshape(1, d),
      mask.reshape(1, d))


def kernel(x, r_ij, neighbors, neighbor_mask, f_ij,
           W_in2f, b_in2f, W_G,
           W_res1, b_res1, W_res2, b_res2, W_res3, b_res3,
           W_dense, b_dense, mask):
    b, n, d = x.shape
    nbh = neighbors.shape[-1]
    sb = f_ij.shape[-1]
    n_edges = b * n * nbh

    x2 = x.reshape(b * n, d)
    y = _tc_pre(x2, W_in2f, b_in2f, blk=1000)

    n_per_w = n_edges // _NW
    nbr3 = neighbors.reshape(_NW, n_per_w // _CHUNK, _CHUNK)
    G = _sc_gather(nbr3, y, n_edges, d)

    f3 = f_ij.reshape(b * n, nbh, sb)
    nm2 = neighbor_mask.reshape(n_edges, 1)
    out = _tc_main(f3, G, nm2, y, x2, W_G,
                   W_res1, b_res1, W_res2, b_res2, W_res3, b_res3,
                   W_dense, b_dense, mask, blk=400)
    return out.reshape(b, n, d)


# speedup vs baseline: 1139.9960x; 1.2799x over previous
import functools

import jax
import jax.numpy as jnp
from jax import lax
from jax.experimental import pallas as pl
from jax.experimental.pallas import tpu as pltpu
from jax.experimental.pallas import tpu_sc as plsc

_LOG2 = 0.6931471805599453


def _ssp(v):
    return jnp.maximum(v, 0.0) + jnp.log1p(jnp.exp(-jnp.abs(v))) - _LOG2


def _tc_pre_body(x_ref, w_ref, b_ref, y_ref):
    v = _ssp(x_ref[...])
    v = jnp.dot(v, w_ref[...], preferred_element_type=jnp.float32) + b_ref[...]
    y_ref[...] = _ssp(v)


def _tc_pre(x2, W_in2f, b_in2f, blk):
    n, d = x2.shape
    grid = (n // blk,)
    return pl.pallas_call(
        _tc_pre_body,
        grid=grid,
        in_specs=[
            pl.BlockSpec((blk, d), lambda i: (i, 0)),
            pl.BlockSpec((d, d), lambda i: (0, 0)),
            pl.BlockSpec((1, d), lambda i: (0, 0)),
        ],
        out_specs=pl.BlockSpec((blk, d), lambda i: (i, 0)),
        out_shape=jax.ShapeDtypeStruct((n, d), jnp.float32),
    )(x2, W_in2f, b_in2f.reshape(1, d))


_NC, _NS = 2, 16
_NW = _NC * _NS
_CHUNK = 80


def _sc_gather(nbr3, y, n_edges, d):
    n_per_w = n_edges // _NW
    n_chunks = n_per_w // _CHUNK
    mesh = plsc.VectorSubcoreMesh(core_axis_name="c", subcore_axis_name="s")

    @functools.partial(
        pl.kernel,
        mesh=mesh,
        out_type=jax.ShapeDtypeStruct((n_edges, d), jnp.float32),
        scratch_types=[
            pltpu.VMEM((n_chunks, _CHUNK), jnp.int32),
            pltpu.VMEM((2, _CHUNK, d), jnp.float32),
            pltpu.SemaphoreType.DMA,
            pltpu.SemaphoreType.DMA,
        ],
    )
    def gather_k(nbr_hbm, y_hbm, out_hbm, idx_v, buf_v, sem_g, sem_w):
        wid = lax.axis_index("s") * _NC + lax.axis_index("c")
        base = wid * n_per_w
        pltpu.sync_copy(nbr_hbm.at[wid], idx_v)
        pltpu.async_copy(y_hbm.at[idx_v.at[0]], buf_v.at[0], sem_g)

        def body(i, carry):
            cur = lax.rem(i, 2)
            nxt = 1 - cur
            pltpu.make_async_copy(
                y_hbm.at[idx_v.at[i]], buf_v.at[cur], sem_g).wait()

            @pl.when(i + 1 < n_chunks)
            def _start_next():
                @pl.when(i >= 1)
                def _drain_prev_write():
                    pltpu.make_async_copy(
                        buf_v.at[nxt],
                        out_hbm.at[pl.ds(base, _CHUNK)], sem_w).wait()

                pltpu.async_copy(
                    y_hbm.at[idx_v.at[i + 1]], buf_v.at[nxt], sem_g)

            pltpu.async_copy(
                buf_v.at[cur],
                out_hbm.at[pl.ds(base + i * _CHUNK, _CHUNK)], sem_w)
            return carry

        lax.fori_loop(0, n_chunks, body, 0)
        pltpu.make_async_copy(
            buf_v.at[0], out_hbm.at[pl.ds(base, _CHUNK)], sem_w).wait()
        pltpu.make_async_copy(
            buf_v.at[1], out_hbm.at[pl.ds(base, _CHUNK)], sem_w).wait()

    return gather_k(nbr3, y)


def _tc_main_body(f_ref, g_ref, nm_ref, y_ref, x_ref,
                  wg_ref, w1_ref, b1_ref, w2_ref, b2_ref, w3_ref, b3_ref,
                  wd_ref, bd_ref, mask_ref, o_ref, *, blk, nbh):
    d = y_ref.shape[-1]
    sb = wg_ref.shape[0]
    f2 = f_ref[...].reshape(blk * nbh, sb)
    wf = jnp.dot(f2, wg_ref[...], preferred_element_type=jnp.float32)
    prod = (g_ref[...] * wf).reshape(blk, nbh, d)
    nm = nm_ref[...].reshape(blk, nbh)
    y2 = jnp.sum(prod * nm[..., None], axis=1)
    y = y_ref[...] + y2
    h = y
    for w_r, b_r in ((w1_ref, b1_ref), (w2_ref, b2_ref), (w3_ref, b3_ref)):
        h = _ssp(h)
        h = jnp.dot(h, w_r[...], preferred_element_type=jnp.float32) + b_r[...]
    y = y + h
    y = _ssp(y)
    y = jnp.dot(y, wd_ref[...], preferred_element_type=jnp.float32) + bd_ref[...]
    o_ref[...] = y + mask_ref[...] * x_ref[...]


def _tc_main(f3, G, nm2, y, x2, W_G,
             W_res1, b_res1, W_res2, b_res2, W_res3, b_res3,
             W_dense, b_dense, mask, blk):
    n, d = x2.shape
    nbh = f3.shape[1]
    sb = f3.shape[2]
    grid = (n // blk,)
    w_spec = pl.BlockSpec((d, d), lambda i: (0, 0))
    b_spec = pl.BlockSpec((1, d), lambda i: (0, 0))
    return pl.pallas_call(
        functools.partial(_tc_main_body, blk=blk, nbh=nbh),
        grid=grid,
        in_specs=[
            pl.BlockSpec((blk, nbh, sb), lambda i: (i, 0, 0)),
            pl.BlockSpec((blk * nbh, d), lambda i: (i, 0)),
            pl.BlockSpec((1, blk, nbh), lambda i: (0, i, 0)),
            pl.BlockSpec((blk, d), lambda i: (i, 0)),
            pl.BlockSpec((blk, d), lambda i: (i, 0)),
            pl.BlockSpec((sb, d), lambda i: (0, 0)),
            w_spec, b_spec, w_spec, b_spec, w_spec, b_spec,
            w_spec, b_spec, b_spec,
        ],
        out_specs=pl.BlockSpec((blk, d), lambda i: (i, 0)),
        out_shape=jax.ShapeDtypeStruct((n, d), jnp.float32),
    )(f3, G, nm2, y, x2, W_G,
      W_res1, b_res1.reshape(1, d), W_res2, b_res2.reshape(1, d),
      W_res3, b_res3.reshape(1, d), W_dense, b_dense.reshape(1, d),
      mask.reshape(1, d))


def kernel(x, r_ij, neighbors, neighbor_mask, f_ij,
           W_in2f, b_in2f, W_G,
           W_res1, b_res1, W_res2, b_res2, W_res3, b_res3,
           W_dense, b_dense, mask):
    b, n, d = x.shape
    nbh = neighbors.shape[-1]
    sb = f_ij.shape[-1]
    n_edges = b * n * nbh

    x2 = x.reshape(b * n, d)
    y = _tc_pre(x2, W_in2f, b_in2f, blk=1000)

    n_per_w = n_edges // _NW
    nbr3 = neighbors.reshape(_NW, n_per_w // _CHUNK, _CHUNK)
    G = _sc_gather(nbr3, y, n_edges, d)

    f3 = f_ij.reshape(b * n, nbh, sb)
    out = _tc_main(f3, G, neighbor_mask, y, x2, W_G,
                   W_res1, b_res1, W_res2, b_res2, W_res3, b_res3,
                   W_dense, b_dense, mask, blk=400)
    return out.reshape(b, n, d)
